# Initial kernel scaffold; baseline (speedup 1.0000x reference)
import jax
import jax.numpy as jnp
from jax.experimental import pallas as pl


def kernel(x, sup0_idx, sup0_val, kernel0, sup1_idx, sup1_val, kernel1, sup2_idx, sup2_val, kernel2, sup3_idx, sup3_val, kernel3, bias):
    raise NotImplementedError("write your pallas kernel here")


# speedup vs baseline: 1.0007x; 1.0007x over previous
import functools

import jax
import jax.numpy as jnp
from jax import lax
from jax.experimental import pallas as pl

N = 10000
E = 160000
F_IN = 256
F_OUT = 256
B = 4
K = 4

NB = 1000


def _mm_body(x_ref, w_ref, o_ref):
    w = w_ref[0]
    for b in range(B):
        o_ref[0, :, b * F_OUT:(b + 1) * F_OUT] = jnp.dot(
            x_ref[b], w, preferred_element_type=jnp.float32)


def _dense_phase(x, w_all):
    grid = (K, N // NB)
    return pl.pallas_call(
        _mm_body,
        grid=grid,
        in_specs=[
            pl.BlockSpec((B, NB, F_IN), lambda i, n: (0, n, 0)),
            pl.BlockSpec((1, F_IN, F_OUT), lambda i, n: (i, 0, 0)),
        ],
        out_specs=pl.BlockSpec((1, NB, B * F_OUT), lambda i, n: (i, n, 0)),
        out_shape=jax.ShapeDtypeStruct((K, N, B * F_OUT), jnp.float32),
    )(x, w_all)


def kernel(x, sup0_idx, sup0_val, kernel0, sup1_idx, sup1_val, kernel1,
           sup2_idx, sup2_val, kernel2, sup3_idx, sup3_val, kernel3, bias):
    w_all = jnp.stack([kernel0, kernel1, kernel2, kernel3])
    res = _dense_phase(x, w_all)

    out = jnp.zeros((N, B * F_OUT), jnp.float32)
    for i, idx in enumerate((sup0_idx, sup1_idx, sup2_idx, sup3_idx)):
        val = (sup0_val, sup1_val, sup2_val, sup3_val)[i]
        gathered = res[i][idx[1]] * val[:, None]
        out = out + jax.ops.segment_sum(gathered, idx[0], num_segments=N)

    out = out.reshape(N, B, F_OUT).transpose(1, 0, 2) + bias
    return out
